# baseline (speedup 1.0000x reference)
import functools

import jax
import jax.numpy as jnp
from jax import lax
from jax.experimental import pallas as pl
from jax.experimental.pallas import tpu as pltpu
from jax.experimental.pallas import tpu_sc as plsc

_NUM_SPECIAL = 4
_NUM_FIXED = 100000
_NUM_LEARNED = 100000
_D = 64
_BATCH = 4096
_HIST = 50
_B = _BATCH * _HIST
_HIST_PAD = 56
_ROWS = _NUM_FIXED + 1

_NC = 2
_NS = 16
_NW = _NC * _NS
_PER_W = _B // _NW
_CH = 128
_NCH = _PER_W // _CH
_NG = _NCH // 2

_LEARNED_START = _NUM_SPECIAL + _NUM_FIXED

_BLK = 8192
_NBF = -(-_ROWS // _BLK)
_L_OFF = _NBF * _BLK
_CROWS = 2 * _L_OFF


_BROWS = _L_OFF // _NW * 2
_BCH = 512
_BNCH = _BROWS // _BCH
_NFULL = _NW // 2 - 1
_TAIL = _ROWS - _NFULL * _BROWS


def _build_body(fixed_hbm, learned_hbm, comb_hbm,
                in0, in1, r0f_v, r0l_v, semi0, semi1, semo0, semo1):
    cid = lax.axis_index("c")
    sid = lax.axis_index("s")
    wid = sid * _NC + cid
    is_f = wid < _NW // 2
    t = jnp.where(is_f, wid, wid - _NW // 2)
    src_base = t * _BROWS
    dst_base = jnp.where(is_f, 0, _L_OFF) + t * _BROWS

    ins = [in0, in1]
    semi = [semi0, semi1]
    semo = [semo0, semo1]

    pltpu.sync_copy(fixed_hbm.at[pl.ds(0, 8)], r0f_v)
    pltpu.sync_copy(learned_hbm.at[pl.ds(0, 8)], r0l_v)
    r0 = [
        jnp.where(is_f, r0l_v[0, pl.ds(k * 16, 16)], r0f_v[0, pl.ds(k * 16, 16)])
        for k in range(_D // 16)
    ]

    def add_rows(buf, nrows, unroll=1):
        def body(i, _):
            for u in range(unroll):
                r = i * unroll + u
                for k in range(_D // 16):
                    plsc.addupdate(buf.at[r, pl.ds(k * 16, 16)], r0[k])
            return 0
        lax.fori_loop(0, nrows // unroll, body, 0)

    def process(tab_hbm):
        @pl.when(t < _NFULL)
        def _():
            def fire_in(c, b):
                pltpu.async_copy(
                    tab_hbm.at[pl.ds(src_base + c * _BCH, _BCH)],
                    ins[b], semi[b])

            def wait_in(c, b):
                pltpu.make_async_copy(
                    tab_hbm.at[pl.ds(src_base + c * _BCH, _BCH)],
                    ins[b], semi[b]).wait()

            def fire_out(c, b):
                pltpu.async_copy(
                    ins[b], comb_hbm.at[pl.ds(dst_base + c * _BCH, _BCH)],
                    semo[b])

            def wait_out(c, b):
                pltpu.make_async_copy(
                    ins[b], comb_hbm.at[pl.ds(dst_base + c * _BCH, _BCH)],
                    semo[b]).wait()

            fire_in(0, 0)
            for c in range(_BNCH):
                b = c % 2
                if c + 1 < _BNCH:
                    if c >= 1:
                        wait_out(c - 1, (c + 1) % 2)
                    fire_in(c + 1, (c + 1) % 2)
                wait_in(c, b)
                add_rows(ins[b], _BCH, unroll=8)
                fire_out(c, b)
            wait_out(_BNCH - 2, _BNCH % 2)
            wait_out(_BNCH - 1, (_BNCH - 1) % 2)

        @pl.when(t == _NFULL)
        def _():
            pltpu.sync_copy(
                tab_hbm.at[pl.ds(src_base, _TAIL)], in0.at[pl.ds(0, _TAIL)])
            add_rows(in0, _TAIL)
            pltpu.sync_copy(
                in0.at[pl.ds(0, _TAIL)],
                comb_hbm.at[pl.ds(dst_base, _TAIL)])

    @pl.when(is_f)
    def _():
        process(fixed_hbm)

    @pl.when(jnp.logical_not(is_f))
    def _():
        process(learned_hbm)


def _build_combined(fixed_table, learned_table):
    mesh = plsc.VectorSubcoreMesh(core_axis_name="c", subcore_axis_name="s")
    return pl.kernel(
        _build_body,
        mesh=mesh,
        compiler_params=pltpu.CompilerParams(use_tc_tiling_on_sc=False),
        out_type=jax.ShapeDtypeStruct((_CROWS, _D), jnp.float32),
        scratch_types=[
            pltpu.VMEM((_BCH, _D), jnp.float32),
            pltpu.VMEM((_BCH, _D), jnp.float32),
            pltpu.VMEM((8, _D), jnp.float32),
            pltpu.VMEM((8, _D), jnp.float32),
            pltpu.SemaphoreType.DMA,
            pltpu.SemaphoreType.DMA,
            pltpu.SemaphoreType.DMA,
            pltpu.SemaphoreType.DMA,
        ],
    )(fixed_table, learned_table)


_GRP = 400
_GB = _GRP // _HIST
_NGRP = _PER_W // _GRP
_GCH = (128, 128, 128, 16)


def _gather_body(ids_hbm, comb_hbm, out_hbm,
                 ids_v, idx0, idx1, rows0, rows1,
                 semg0, semg1, semo0, semo1):
    cid = lax.axis_index("c")
    sid = lax.axis_index("s")
    wid = sid * _NC + cid
    base = wid * _PER_W
    bat0 = wid * (_PER_W // _HIST)

    idx = [idx0, idx1]
    rows = [rows0, rows1]
    semg = [semg0, semg1]
    semo = [semo0, semo1]

    pltpu.sync_copy(ids_hbm.at[pl.ds(base, _PER_W)], ids_v)

    def fire(g, b):
        for k in range(_GRP // 16):
            sl = pl.ds(k * 16, 16)
            idv = ids_v[pl.ds(g * _GRP + k * 16, 16)]
            is_l = idv >= _LEARNED_START
            fi = jnp.maximum(idv - (_NUM_SPECIAL - 1), 0)
            ci = jnp.where(is_l, idv + (_L_OFF - (_LEARNED_START - 1)), fi)
            idx[b][sl] = ci
        off = 0
        for n in _GCH:
            pltpu.async_copy(
                comb_hbm.at[idx[b].at[pl.ds(off, n)]],
                rows[b].at[pl.ds(off, n)], semg[b])
            off += n

    def wait_gathers(b):
        off = 0
        for n in _GCH:
            pltpu.make_async_copy(
                comb_hbm.at[idx[b].at[pl.ds(off, n)]],
                rows[b].at[pl.ds(off, n)], semg[b]).wait()
            off += n

    def put_outs(g, b):
        for j in range(_GB):
            pltpu.async_copy(
                rows[b].at[pl.ds(j * _HIST, _HIST)],
                out_hbm.at[bat0 + g * _GB + j], semo[b])

    def wait_outs(g, b):
        for j in range(_GB):
            pltpu.make_async_copy(
                rows[b].at[pl.ds(j * _HIST, _HIST)],
                out_hbm.at[bat0 + g * _GB + j], semo[b]).wait()

    fire(0, 0)
    for g in range(_NGRP):
        b = g % 2
        if g + 1 < _NGRP:
            if g >= 1:
                wait_outs(g - 1, (g + 1) % 2)
            fire(g + 1, (g + 1) % 2)
        wait_gathers(b)
        put_outs(g, b)
    wait_outs(_NGRP - 2, _NGRP % 2)
    wait_outs(_NGRP - 1, (_NGRP - 1) % 2)


@jax.jit
def _emb(ids_flat, fixed_table, learned_table):
    comb = _build_combined(fixed_table, learned_table)
    mesh = plsc.VectorSubcoreMesh(core_axis_name="c", subcore_axis_name="s")
    out = pl.kernel(
        _gather_body,
        mesh=mesh,
        compiler_params=pltpu.CompilerParams(use_tc_tiling_on_sc=False),
        out_type=jax.ShapeDtypeStruct((_BATCH, _HIST, _D), jnp.float32),
        scratch_types=[
            pltpu.VMEM((_PER_W,), jnp.int32),
            pltpu.VMEM((_GRP,), jnp.int32),
            pltpu.VMEM((_GRP,), jnp.int32),
            pltpu.VMEM((_GRP, _D), jnp.float32),
            pltpu.VMEM((_GRP, _D), jnp.float32),
            pltpu.SemaphoreType.DMA,
            pltpu.SemaphoreType.DMA,
            pltpu.SemaphoreType.DMA,
            pltpu.SemaphoreType.DMA,
        ],
    )(ids_flat, comb)
    return out


def kernel(ids_tensor, fixed_table, learned_table):
    ids_flat = ids_tensor.reshape(_B)
    return _emb(ids_flat, fixed_table, learned_table)


# speedup vs baseline: 1.0034x; 1.0034x over previous
import jax
import jax.numpy as jnp
from jax import lax
from jax.experimental import pallas as pl
from jax.experimental.pallas import tpu as pltpu
from jax.experimental.pallas import tpu_sc as plsc

_NUM_SPECIAL = 4
_NUM_FIXED = 100000
_NUM_LEARNED = 100000
_D = 64
_BATCH = 4096
_HIST = 50
_B = _BATCH * _HIST
_ROWS = _NUM_FIXED + 1

_NC = 2
_NS = 16
_NW = _NC * _NS
_PER_W = _B // _NW

_LEARNED_START = _NUM_SPECIAL + _NUM_FIXED

_L_OFF = 106496
_CROWS = 2 * _L_OFF


_BROWS = _L_OFF // _NW * 2
_BCH = 512
_BNCH = _BROWS // _BCH
_NFULL = _NW // 2 - 1
_TAIL = _ROWS - _NFULL * _BROWS


def _build_body(fixed_hbm, learned_hbm, comb_hbm,
                in0, in1, r0f_v, r0l_v, semi0, semi1, semo0, semo1):
    cid = lax.axis_index("c")
    sid = lax.axis_index("s")
    wid = sid * _NC + cid
    is_f = wid < _NW // 2
    t = jnp.where(is_f, wid, wid - _NW // 2)
    src_base = t * _BROWS
    dst_base = jnp.where(is_f, 0, _L_OFF) + t * _BROWS

    ins = [in0, in1]
    semi = [semi0, semi1]
    semo = [semo0, semo1]

    pltpu.sync_copy(fixed_hbm.at[pl.ds(0, 8)], r0f_v)
    pltpu.sync_copy(learned_hbm.at[pl.ds(0, 8)], r0l_v)
    r0 = [
        jnp.where(is_f, r0l_v[0, pl.ds(k * 16, 16)], r0f_v[0, pl.ds(k * 16, 16)])
        for k in range(_D // 16)
    ]

    def add_rows(buf, nrows, unroll=1):
        def body(i, _):
            for u in range(unroll):
                r = i * unroll + u
                for k in range(_D // 16):
                    plsc.addupdate(buf.at[r, pl.ds(k * 16, 16)], r0[k])
            return 0
        lax.fori_loop(0, nrows // unroll, body, 0)

    def process(tab_hbm):
        @pl.when(t < _NFULL)
        def _():
            def fire_in(c, b):
                pltpu.async_copy(
                    tab_hbm.at[pl.ds(src_base + c * _BCH, _BCH)],
                    ins[b], semi[b])

            def wait_in(c, b):
                pltpu.make_async_copy(
                    tab_hbm.at[pl.ds(src_base + c * _BCH, _BCH)],
                    ins[b], semi[b]).wait()

            def fire_out(c, b):
                pltpu.async_copy(
                    ins[b], comb_hbm.at[pl.ds(dst_base + c * _BCH, _BCH)],
                    semo[b])

            def wait_out(c, b):
                pltpu.make_async_copy(
                    ins[b], comb_hbm.at[pl.ds(dst_base + c * _BCH, _BCH)],
                    semo[b]).wait()

            fire_in(0, 0)
            for c in range(_BNCH):
                b = c % 2
                if c + 1 < _BNCH:
                    if c >= 1:
                        wait_out(c - 1, (c + 1) % 2)
                    fire_in(c + 1, (c + 1) % 2)
                wait_in(c, b)
                add_rows(ins[b], _BCH, unroll=4)
                fire_out(c, b)
            wait_out(_BNCH - 2, _BNCH % 2)
            wait_out(_BNCH - 1, (_BNCH - 1) % 2)

        @pl.when(t == _NFULL)
        def _():
            pltpu.sync_copy(
                tab_hbm.at[pl.ds(src_base, _TAIL)], in0.at[pl.ds(0, _TAIL)])
            add_rows(in0, _TAIL)
            pltpu.sync_copy(
                in0.at[pl.ds(0, _TAIL)],
                comb_hbm.at[pl.ds(dst_base, _TAIL)])

    @pl.when(is_f)
    def _():
        process(fixed_hbm)

    @pl.when(jnp.logical_not(is_f))
    def _():
        process(learned_hbm)


def _build_combined(fixed_table, learned_table):
    mesh = plsc.VectorSubcoreMesh(core_axis_name="c", subcore_axis_name="s")
    return pl.kernel(
        _build_body,
        mesh=mesh,
        compiler_params=pltpu.CompilerParams(use_tc_tiling_on_sc=False),
        out_type=jax.ShapeDtypeStruct((_CROWS, _D), jnp.float32),
        scratch_types=[
            pltpu.VMEM((_BCH, _D), jnp.float32),
            pltpu.VMEM((_BCH, _D), jnp.float32),
            pltpu.VMEM((8, _D), jnp.float32),
            pltpu.VMEM((8, _D), jnp.float32),
            pltpu.SemaphoreType.DMA,
            pltpu.SemaphoreType.DMA,
            pltpu.SemaphoreType.DMA,
            pltpu.SemaphoreType.DMA,
        ],
    )(fixed_table, learned_table)


_GRP = 400
_GB = _GRP // _HIST
_NGRP = _PER_W // _GRP
_GCH = (128, 128, 128, 16)


def _gather_body(ids_hbm, comb_hbm, out_hbm,
                 ids_v, idx0, idx1, rows0, rows1,
                 semg0, semg1, semo0, semo1):
    cid = lax.axis_index("c")
    sid = lax.axis_index("s")
    wid = sid * _NC + cid
    base = wid * _PER_W
    bat0 = wid * (_PER_W // _HIST)

    idx = [idx0, idx1]
    rows = [rows0, rows1]
    semg = [semg0, semg1]
    semo = [semo0, semo1]

    pltpu.sync_copy(ids_hbm.at[pl.ds(base, _PER_W)], ids_v)

    def fire(g, b):
        for k in range(_GRP // 16):
            sl = pl.ds(k * 16, 16)
            idv = ids_v[pl.ds(g * _GRP + k * 16, 16)]
            is_l = idv >= _LEARNED_START
            fi = jnp.maximum(idv - (_NUM_SPECIAL - 1), 0)
            ci = jnp.where(is_l, idv + (_L_OFF - (_LEARNED_START - 1)), fi)
            idx[b][sl] = ci
        off = 0
        for n in _GCH:
            pltpu.async_copy(
                comb_hbm.at[idx[b].at[pl.ds(off, n)]],
                rows[b].at[pl.ds(off, n)], semg[b])
            off += n

    def wait_gathers(b):
        off = 0
        for n in _GCH:
            pltpu.make_async_copy(
                comb_hbm.at[idx[b].at[pl.ds(off, n)]],
                rows[b].at[pl.ds(off, n)], semg[b]).wait()
            off += n

    def put_outs(g, b):
        for j in range(_GB):
            pltpu.async_copy(
                rows[b].at[pl.ds(j * _HIST, _HIST)],
                out_hbm.at[bat0 + g * _GB + j], semo[b])

    def wait_outs(g, b):
        for j in range(_GB):
            pltpu.make_async_copy(
                rows[b].at[pl.ds(j * _HIST, _HIST)],
                out_hbm.at[bat0 + g * _GB + j], semo[b]).wait()

    fire(0, 0)
    for g in range(_NGRP):
        b = g % 2
        if g + 1 < _NGRP:
            if g >= 1:
                wait_outs(g - 1, (g + 1) % 2)
            fire(g + 1, (g + 1) % 2)
        wait_gathers(b)
        put_outs(g, b)
    wait_outs(_NGRP - 2, _NGRP % 2)
    wait_outs(_NGRP - 1, (_NGRP - 1) % 2)


@jax.jit
def _emb(ids_flat, fixed_table, learned_table):
    comb = _build_combined(fixed_table, learned_table)
    mesh = plsc.VectorSubcoreMesh(core_axis_name="c", subcore_axis_name="s")
    out = pl.kernel(
        _gather_body,
        mesh=mesh,
        compiler_params=pltpu.CompilerParams(use_tc_tiling_on_sc=False),
        out_type=jax.ShapeDtypeStruct((_BATCH, _HIST, _D), jnp.float32),
        scratch_types=[
            pltpu.VMEM((_PER_W,), jnp.int32),
            pltpu.VMEM((_GRP,), jnp.int32),
            pltpu.VMEM((_GRP,), jnp.int32),
            pltpu.VMEM((_GRP, _D), jnp.float32),
            pltpu.VMEM((_GRP, _D), jnp.float32),
            pltpu.SemaphoreType.DMA,
            pltpu.SemaphoreType.DMA,
            pltpu.SemaphoreType.DMA,
            pltpu.SemaphoreType.DMA,
        ],
    )(ids_flat, comb)
    return out


def kernel(ids_tensor, fixed_table, learned_table):
    ids_flat = ids_tensor.reshape(_B)
    return _emb(ids_flat, fixed_table, learned_table)
